# baseline (speedup 1.0000x reference)
import functools

import jax
import jax.numpy as jnp
from jax import lax
from jax.experimental import pallas as pl
from jax.experimental.pallas import tpu as pltpu
from jax.experimental.pallas import tpu_sc as plsc

_BB = 128
_N = 8192
_B = 4096
_ROWS_PER_W = 128


def _normalize_quantized(d):
    s = d[:, 0:1] * d[:, 0:1] + d[:, 1:2] * d[:, 1:2] + d[:, 2:3] * d[:, 2:3]
    norm = s * lax.rsqrt(s)
    rn = pl.reciprocal(norm, approx=True)
    return (d * rn).astype(jnp.bfloat16).astype(jnp.float32)


def _argmax_body(d0_ref, d1_ref, e_ref, idx0_ref, idx1_ref):
    e = e_ref[...]
    dn0 = _normalize_quantized(d0_ref[...])
    dn1 = _normalize_quantized(d1_ref[...])
    cdims = (((1,), (1,)), ((), ()))
    dist0 = lax.dot_general(e, dn0, cdims, preferred_element_type=jnp.float32)
    dist1 = lax.dot_general(e, dn1, cdims, preferred_element_type=jnp.float32)
    iota = lax.broadcasted_iota(jnp.int32, dist0.shape, 0)
    m0 = jnp.max(dist0, axis=0, keepdims=True)
    m1 = jnp.max(dist1, axis=0, keepdims=True)
    i0 = jnp.min(jnp.where(dist0 == m0, iota, _N), axis=0)
    i1 = jnp.min(jnp.where(dist1 == m1, iota, _N), axis=0)
    idx0_ref[...] = i0[None, None, :]
    idx1_ref[...] = i1[None, None, :]


def _tc_argmax(d0, d1, endpoints):
    nb = _B // _BB
    return pl.pallas_call(
        _argmax_body,
        grid=(nb,),
        in_specs=[
            pl.BlockSpec((_BB, 3), lambda i: (i, 0)),
            pl.BlockSpec((_BB, 3), lambda i: (i, 0)),
            pl.BlockSpec((_N, 3), lambda i: (0, 0)),
        ],
        out_specs=[
            pl.BlockSpec((1, 1, _BB), lambda i: (i, 0, 0)),
            pl.BlockSpec((1, 1, _BB), lambda i: (i, 0, 0)),
        ],
        out_shape=[
            jax.ShapeDtypeStruct((nb, 1, _BB), jnp.int32),
            jax.ShapeDtypeStruct((nb, 1, _BB), jnp.int32),
        ],
    )(d0, d1, endpoints)


_RPB = 2
_LAG = 8
_CHW = _RPB * 2 * _N


def _sc_fill(idx0, idx1, zrows):
    mesh = plsc.VectorSubcoreMesh(core_axis_name="c", subcore_axis_name="s")
    info = plsc.get_sparse_core_info()
    nc = info.num_cores
    ns = info.num_subcores
    total = _B * 2 * _N
    n_chunks = _ROWS_PER_W // _RPB

    @functools.partial(
        pl.kernel,
        mesh=mesh,
        out_type=jax.ShapeDtypeStruct((total,), jnp.float32),
        scratch_types=[
            pltpu.VMEM((1, _BB), jnp.int32),
            pltpu.VMEM((1, _BB), jnp.int32),
            pltpu.VMEM((_CHW,), jnp.float32),
            pltpu.VMEM((2, _BB), jnp.int32),
            pltpu.VMEM((2, _BB), jnp.float32),
            pltpu.VMEM_SHARED((ns * _CHW,), jnp.float32),
            pltpu.SemaphoreType.DMA,
            pltpu.SemaphoreType.DMA,
        ],
        compiler_params=pltpu.CompilerParams(needs_layout_passes=False),
    )
    def fill(idx0_hbm, idx1_hbm, zrows_hbm, out_hbm,
             idx0_v, idx1_v, zbuf, pos_v, ones_v, shared, sem, sem2):
        sid = lax.axis_index("s")
        w = sid * nc + lax.axis_index("c")
        pltpu.sync_copy(idx0_hbm.at[w], idx0_v)
        pltpu.sync_copy(idx1_hbm.at[w], idx1_v)
        pltpu.sync_copy(zrows_hbm, zbuf)
        myslot = shared.at[pl.ds(sid * _CHW, _CHW)]
        pltpu.sync_copy(zbuf, myslot)
        base = w * _ROWS_PER_W * 2 * _N

        def fire(i, _):
            pltpu.make_async_copy(
                myslot, out_hbm.at[pl.ds(base + i * _CHW, _CHW)], sem).start()

            @pl.when(i >= _LAG)
            def _():
                pltpu.make_async_copy(
                    myslot, out_hbm.at[pl.ds(0, _CHW)], sem).wait()

            return 0

        lax.fori_loop(0, n_chunks, fire, 0)

        def drain(i, _):
            pltpu.make_async_copy(
                myslot, out_hbm.at[pl.ds(0, _CHW)], sem).wait()
            return 0

        lax.fori_loop(0, _LAG, drain, 0)

        lanes = lax.iota(jnp.int32, 16)
        one16 = jnp.ones((16,), jnp.float32)
        for c in range(_ROWS_PER_W // 16):
            p0 = idx0_v[0, pl.ds(c * 16, 16)]
            p1 = idx1_v[0, pl.ds(c * 16, 16)]
            rowbase = (w * _ROWS_PER_W + c * 16 + lanes) * (2 * _N)
            pos_v[0, pl.ds(c * 16, 16)] = rowbase + p0
            pos_v[1, pl.ds(c * 16, 16)] = rowbase + _N + p1
            ones_v[0, pl.ds(c * 16, 16)] = one16
            ones_v[1, pl.ds(c * 16, 16)] = one16
        pltpu.make_async_copy(
            ones_v.at[0], out_hbm.at[pos_v.at[0]], sem2).start()
        pltpu.make_async_copy(
            ones_v.at[1], out_hbm.at[pos_v.at[1]], sem2).start()
        pltpu.make_async_copy(
            ones_v.at[0], out_hbm.at[pos_v.at[0]], sem2).wait()
        pltpu.make_async_copy(
            ones_v.at[1], out_hbm.at[pos_v.at[1]], sem2).wait()

    return fill(idx0, idx1, zrows)


@jax.jit
def kernel(directions, endpoints):
    b, _, k = directions.shape
    n = endpoints.shape[0]
    d0 = directions[:, :, 0]
    d1 = directions[:, :, 1]
    idx0, idx1 = _tc_argmax(d0, d1, endpoints)
    zrows = jnp.zeros((_CHW,), jnp.float32)
    out = _sc_fill(idx0, idx1, zrows)
    return out.reshape(b, k, n).transpose(0, 2, 1)


# speedup vs baseline: 1.2139x; 1.2139x over previous
import functools

import jax
import jax.numpy as jnp
from jax.experimental import pallas as pl
from jax.experimental.pallas import tpu as pltpu

_BB = 256
_NT = 8192


def _normalize_quantized(d):
    s = d[:, 0:1] * d[:, 0:1] + d[:, 1:2] * d[:, 1:2] + d[:, 2:3] * d[:, 2:3]
    norm = s * jax.lax.rsqrt(s)
    rn = pl.reciprocal(norm, approx=True)
    return (d * rn).astype(jnp.bfloat16).astype(jnp.float32)


def _body(d0_ref, d1_ref, et_ref, out_ref, idx0_ref, idx1_ref):
    nn = pl.program_id(1)

    @pl.when(nn == 0)
    def _compute_argmax():
        et = et_ref[...]
        n = et.shape[1]
        dn0 = _normalize_quantized(d0_ref[...])
        dn1 = _normalize_quantized(d1_ref[...])
        dist0 = jnp.dot(dn0, et, preferred_element_type=jnp.float32)
        dist1 = jnp.dot(dn1, et, preferred_element_type=jnp.float32)
        iota = jax.lax.broadcasted_iota(jnp.int32, dist0.shape, 1)
        m0 = jnp.max(dist0, axis=1, keepdims=True)
        m1 = jnp.max(dist1, axis=1, keepdims=True)
        i0 = jnp.min(jnp.where(dist0 == m0, iota, n), axis=1, keepdims=True)
        i1 = jnp.min(jnp.where(dist1 == m1, iota, n), axis=1, keepdims=True)
        idx0_ref[...] = jnp.broadcast_to(i0, idx0_ref.shape)
        idx1_ref[...] = jnp.broadcast_to(i1, idx1_ref.shape)

    n = et_ref.shape[1]
    base = nn * _NT
    col = jax.lax.broadcasted_iota(jnp.int32, (_BB, _NT), 1)
    t = jnp.where(base >= n, idx1_ref[:, 0:1] + n, idx0_ref[:, 0:1])
    out_ref[...] = jnp.where(col == (t - base), 1.0, 0.0).astype(jnp.float32)


@jax.jit
def kernel(directions, endpoints):
    b, _, k = directions.shape
    n = endpoints.shape[0]
    d0 = directions[:, :, 0]
    d1 = directions[:, :, 1]
    et = endpoints.T
    grid = (b // _BB, (n * k) // _NT)
    out = pl.pallas_call(
        _body,
        grid=grid,
        in_specs=[
            pl.BlockSpec((_BB, 3), lambda nb, nn: (nb, 0)),
            pl.BlockSpec((_BB, 3), lambda nb, nn: (nb, 0)),
            pl.BlockSpec((3, n), lambda nb, nn: (0, 0)),
        ],
        out_specs=pl.BlockSpec((_BB, _NT), lambda nb, nn: (nb, nn)),
        out_shape=jax.ShapeDtypeStruct((b, n * k), jnp.float32),
        scratch_shapes=[
            pltpu.VMEM((_BB, 128), jnp.int32),
            pltpu.VMEM((_BB, 128), jnp.int32),
        ],
        compiler_params=pltpu.CompilerParams(
            dimension_semantics=("parallel", "arbitrary"),
        ),
    )(d0, d1, et)
    return out.reshape(b, k, n).transpose(0, 2, 1)
